# baseline (speedup 1.0000x reference)
import functools

import jax
import jax.numpy as jnp
from jax.experimental import pallas as pl
from jax.experimental.pallas import tpu as pltpu

T = 2048
D = 1024
E = 8
HRP = 640
L = 1024
CLAP = 512
TT = 1024
NT = T // TT

_PREC = jax.lax.Precision.DEFAULT


def _dot(a, b):
    return jnp.dot(a, b, precision=_PREC, preferred_element_type=jnp.float32)


def _body(x_ref, Wr1_ref, br1_ref, Wr2_ref, br2_ref, We1_ref, be1_ref,
          We2_ref, be2_ref, Wm_ref, Wo_ref, bo_ref, out_ref,
          ws_ref, wmo_ref, be2p_ref):
    e = pl.program_id(1)

    bf = lambda a: a.astype(jnp.bfloat16)

    @pl.when(e == 0)
    def _():
        wmo_ref[:] = bf(_dot(bf(Wm_ref[:]), bf(Wo_ref[:])))
        be2p_ref[:] = _dot(bf(be2_ref[:]), wmo_ref[:])
        h = jnp.maximum(_dot(bf(x_ref[:]), bf(Wr1_ref[:])) + br1_ref[:], 0.0)
        logits = _dot(bf(h), bf(Wr2_ref[:])) + br2_ref[:]
        m = jnp.max(logits, axis=1, keepdims=True)
        p = jnp.exp(logits - m)
        ws = p / jnp.sum(p, axis=1, keepdims=True)
        ws_ref[:] = ws
        out_ref[:] = _dot(ws, be2p_ref[:]) + bo_ref[:]

    w2p = bf(_dot(bf(We2_ref[0]), wmo_ref[:]))
    he = jnp.maximum(_dot(bf(x_ref[:]), bf(We1_ref[0])) + be1_ref[0], 0.0)
    y = _dot(bf(he), w2p)
    onehot = (jax.lax.broadcasted_iota(jnp.int32, (E, CLAP), 0) == e
              ).astype(jnp.bfloat16)
    wcol = _dot(bf(ws_ref[:]), onehot)
    out_ref[:] += wcol * y


@functools.partial(jax.jit)
def kernel(x, W_r1, b_r1, W_r2, b_r2, W_e1, b_e1, W_e2, b_e2,
           W_model, W_out, b_out):
    f32 = jnp.float32
    bf16 = jnp.bfloat16
    hr = W_r1.shape[1]
    pad = HRP - hr
    Wr1p = jnp.pad(W_r1, ((0, 0), (0, pad)))
    br1p = jnp.pad(b_r1, ((0, pad),)).reshape(1, HRP)
    Wr2p = jnp.pad(W_r2, ((0, pad), (0, 0)))
    br2 = b_r2.reshape(1, E)
    be1 = b_e1.reshape(E, 1, L)
    bo = b_out.reshape(1, CLAP)

    out = pl.pallas_call(
        _body,
        grid=(NT, E),
        in_specs=[
            pl.BlockSpec((TT, D), lambda t, e: (t, 0)),
            pl.BlockSpec((D, HRP), lambda t, e: (0, 0)),
            pl.BlockSpec((1, HRP), lambda t, e: (0, 0)),
            pl.BlockSpec((HRP, E), lambda t, e: (0, 0)),
            pl.BlockSpec((1, E), lambda t, e: (0, 0)),
            pl.BlockSpec((1, D, L), lambda t, e: (e, 0, 0)),
            pl.BlockSpec((1, 1, L), lambda t, e: (e, 0, 0)),
            pl.BlockSpec((1, L, L), lambda t, e: (e, 0, 0)),
            pl.BlockSpec((E, L), lambda t, e: (0, 0)),
            pl.BlockSpec((L, L), lambda t, e: (0, 0)),
            pl.BlockSpec((L, CLAP), lambda t, e: (0, 0)),
            pl.BlockSpec((1, CLAP), lambda t, e: (0, 0)),
        ],
        out_specs=pl.BlockSpec((TT, CLAP), lambda t, e: (t, 0)),
        out_shape=jax.ShapeDtypeStruct((T, CLAP), f32),
        scratch_shapes=[
            pltpu.VMEM((TT, E), f32),
            pltpu.VMEM((L, CLAP), bf16),
            pltpu.VMEM((E, CLAP), f32),
        ],
        compiler_params=pltpu.CompilerParams(
            dimension_semantics=("parallel", "arbitrary")),
    )(x, Wr1p, br1p, Wr2p, br2, W_e1, be1, W_e2, b_e2, W_model, W_out, bo)

    return (out, jnp.zeros((), f32))


# speedup vs baseline: 1.0627x; 1.0627x over previous
import functools

import jax
import jax.numpy as jnp
from jax.experimental import pallas as pl
from jax.experimental.pallas import tpu as pltpu

T = 2048
D = 1024
E = 8
HRP = 640
L = 1024
CLAP = 512
TT = 1024
NT = T // TT

_PREC = jax.lax.Precision.DEFAULT


def _dot(a, b):
    return jnp.dot(a, b, precision=_PREC, preferred_element_type=jnp.float32)


def _body(x_ref, Wr1_ref, br1_ref, Wr2_ref, br2_ref, We1_ref, be1_ref,
          We2_ref, be2_ref, Wm_ref, Wo_ref, bo_ref, out_ref,
          ws_ref, wmo_ref, w2p_ref, be2p_ref, xbf_ref):
    e = pl.program_id(0)
    t = pl.program_id(1)
    rows = pl.ds(t * TT, TT)

    bf = lambda a: a.astype(jnp.bfloat16)

    @pl.when(jnp.logical_and(e == 0, t == 0))
    def _():
        xbf_ref[:] = bf(x_ref[:])
        wmo_ref[:] = bf(_dot(bf(Wm_ref[:]), bf(Wo_ref[:])))
        be2p_ref[:] = _dot(bf(be2_ref[:]), wmo_ref[:])

    @pl.when(t == 0)
    def _():
        w2p_ref[:] = bf(_dot(bf(We2_ref[0]), wmo_ref[:]))

    @pl.when(e == 0)
    def _():
        h = jnp.maximum(bf(_dot(xbf_ref[rows, :], bf(Wr1_ref[:])))
                        + bf(br1_ref[:]), jnp.bfloat16(0.0))
        logits = _dot(h, bf(Wr2_ref[:])) + br2_ref[:]
        m = jnp.max(logits, axis=1, keepdims=True)
        p = jnp.exp(logits - m)
        ws = p / jnp.sum(p, axis=1, keepdims=True)
        ws_ref[rows, :] = ws
        out_ref[rows, :] = _dot(ws, be2p_ref[:]) + bo_ref[:]

    he = jnp.maximum(bf(_dot(xbf_ref[rows, :], bf(We1_ref[0])))
                     + bf(be1_ref[0]), jnp.bfloat16(0.0))
    y = _dot(he, w2p_ref[:])
    onehot = (jax.lax.broadcasted_iota(jnp.int32, (E, CLAP), 0) == e
              ).astype(jnp.bfloat16)
    wcol = _dot(bf(ws_ref[rows, :]), onehot)
    out_ref[rows, :] += wcol * y


@functools.partial(jax.jit)
def kernel(x, W_r1, b_r1, W_r2, b_r2, W_e1, b_e1, W_e2, b_e2,
           W_model, W_out, b_out):
    f32 = jnp.float32
    bf16 = jnp.bfloat16
    hr = W_r1.shape[1]
    pad = HRP - hr
    Wr1p = jnp.pad(W_r1, ((0, 0), (0, pad)))
    br1p = jnp.pad(b_r1, ((0, pad),)).reshape(1, HRP)
    Wr2p = jnp.pad(W_r2, ((0, pad), (0, 0)))
    br2 = b_r2.reshape(1, E)
    be1 = b_e1.reshape(E, 1, L)
    bo = b_out.reshape(1, CLAP)

    out = pl.pallas_call(
        _body,
        grid=(E, NT),
        in_specs=[
            pl.BlockSpec((T, D), lambda e, t: (0, 0)),
            pl.BlockSpec((D, HRP), lambda e, t: (0, 0)),
            pl.BlockSpec((1, HRP), lambda e, t: (0, 0)),
            pl.BlockSpec((HRP, E), lambda e, t: (0, 0)),
            pl.BlockSpec((1, E), lambda e, t: (0, 0)),
            pl.BlockSpec((1, D, L), lambda e, t: (e, 0, 0)),
            pl.BlockSpec((1, 1, L), lambda e, t: (e, 0, 0)),
            pl.BlockSpec((1, L, L), lambda e, t: (e, 0, 0)),
            pl.BlockSpec((E, L), lambda e, t: (0, 0)),
            pl.BlockSpec((L, L), lambda e, t: (0, 0)),
            pl.BlockSpec((L, CLAP), lambda e, t: (0, 0)),
            pl.BlockSpec((1, CLAP), lambda e, t: (0, 0)),
        ],
        out_specs=pl.BlockSpec((T, CLAP), lambda e, t: (0, 0)),
        out_shape=jax.ShapeDtypeStruct((T, CLAP), f32),
        scratch_shapes=[
            pltpu.VMEM((T, E), f32),
            pltpu.VMEM((L, CLAP), bf16),
            pltpu.VMEM((L, CLAP), bf16),
            pltpu.VMEM((E, CLAP), f32),
            pltpu.VMEM((T, D), bf16),
        ],
        compiler_params=pltpu.CompilerParams(
            dimension_semantics=("arbitrary", "arbitrary")),
    )(x, Wr1p, br1p, Wr2p, br2, W_e1, be1, W_e2, b_e2, W_model, W_out, bo)

    return (out, jnp.zeros((), f32))


# speedup vs baseline: 1.0804x; 1.0166x over previous
import functools

import jax
import jax.numpy as jnp
from jax.experimental import pallas as pl
from jax.experimental.pallas import tpu as pltpu

T = 2048
D = 1024
E = 8
HRP = 640
L = 1024
CLAP = 512
TT = 2048
NT = T // TT

_PREC = jax.lax.Precision.DEFAULT


def _dot(a, b):
    return jnp.dot(a, b, precision=_PREC, preferred_element_type=jnp.float32)


def _body(x_ref, Wr1_ref, br1_ref, Wr2_ref, br2_ref, We1_ref, be1_ref,
          We2_ref, be2_ref, Wm_ref, Wo_ref, bo_ref, out_ref,
          ws_ref, wmo_ref, w2p_ref, be2p_ref):
    e = pl.program_id(0)
    t = pl.program_id(1)
    rows = pl.ds(t * TT, TT)

    bf = lambda a: a.astype(jnp.bfloat16)

    @pl.when(jnp.logical_and(e == 0, t == 0))
    def _():
        wmo_ref[:] = bf(_dot(bf(Wm_ref[:]), bf(Wo_ref[:])))
        be2p_ref[:] = _dot(bf(be2_ref[:]), wmo_ref[:])

    @pl.when(t == 0)
    def _():
        w2p_ref[:] = bf(_dot(bf(We2_ref[0]), wmo_ref[:]))

    @pl.when(e == 0)
    def _():
        h = jnp.maximum(_dot(bf(x_ref[rows, :]), bf(Wr1_ref[:])) + br1_ref[:],
                        0.0)
        logits = _dot(bf(h), bf(Wr2_ref[:])) + br2_ref[:]
        m = jnp.max(logits, axis=1, keepdims=True)
        p = jnp.exp(logits - m)
        ws = p / jnp.sum(p, axis=1, keepdims=True)
        ws_ref[rows, :] = ws
        out_ref[rows, :] = _dot(ws, be2p_ref[:]) + bo_ref[:]

    he = jnp.maximum(bf(_dot(bf(x_ref[rows, :]), bf(We1_ref[0])))
                     + bf(be1_ref[0]), jnp.bfloat16(0.0))
    y = _dot(he, w2p_ref[:])
    onehot = (jax.lax.broadcasted_iota(jnp.int32, (E, CLAP), 0) == e
              ).astype(jnp.bfloat16)
    wcol = _dot(bf(ws_ref[rows, :]), onehot)
    out_ref[rows, :] += wcol * y


@functools.partial(jax.jit)
def kernel(x, W_r1, b_r1, W_r2, b_r2, W_e1, b_e1, W_e2, b_e2,
           W_model, W_out, b_out):
    f32 = jnp.float32
    bf16 = jnp.bfloat16
    hr = W_r1.shape[1]
    pad = HRP - hr
    Wr1p = jnp.pad(W_r1, ((0, 0), (0, pad)))
    br1p = jnp.pad(b_r1, ((0, pad),)).reshape(1, HRP)
    Wr2p = jnp.pad(W_r2, ((0, pad), (0, 0)))
    br2 = b_r2.reshape(1, E)
    be1 = b_e1.reshape(E, 1, L)
    bo = b_out.reshape(1, CLAP)

    out = pl.pallas_call(
        _body,
        grid=(E, NT),
        in_specs=[
            pl.BlockSpec((T, D), lambda e, t: (0, 0)),
            pl.BlockSpec((D, HRP), lambda e, t: (0, 0)),
            pl.BlockSpec((1, HRP), lambda e, t: (0, 0)),
            pl.BlockSpec((HRP, E), lambda e, t: (0, 0)),
            pl.BlockSpec((1, E), lambda e, t: (0, 0)),
            pl.BlockSpec((1, D, L), lambda e, t: (e, 0, 0)),
            pl.BlockSpec((1, 1, L), lambda e, t: (e, 0, 0)),
            pl.BlockSpec((1, L, L), lambda e, t: (e, 0, 0)),
            pl.BlockSpec((E, L), lambda e, t: (0, 0)),
            pl.BlockSpec((L, L), lambda e, t: (0, 0)),
            pl.BlockSpec((L, CLAP), lambda e, t: (0, 0)),
            pl.BlockSpec((1, CLAP), lambda e, t: (0, 0)),
        ],
        out_specs=pl.BlockSpec((T, CLAP), lambda e, t: (0, 0)),
        out_shape=jax.ShapeDtypeStruct((T, CLAP), f32),
        scratch_shapes=[
            pltpu.VMEM((T, E), f32),
            pltpu.VMEM((L, CLAP), bf16),
            pltpu.VMEM((L, CLAP), bf16),
            pltpu.VMEM((E, CLAP), f32),
        ],
        compiler_params=pltpu.CompilerParams(
            dimension_semantics=("arbitrary", "arbitrary")),
    )(x, Wr1p, br1p, Wr2p, br2, W_e1, be1, W_e2, b_e2, W_model, W_out, bo)

    return (out, jnp.zeros((), f32))


# speedup vs baseline: 1.1030x; 1.0209x over previous
import functools

import jax
import jax.numpy as jnp
from jax.experimental import pallas as pl
from jax.experimental.pallas import tpu as pltpu

T = 2048
D = 1024
E = 8
HRP = 640
L = 1024
CLAP = 512
TT = 2048
NT = T // TT

_PREC = jax.lax.Precision.DEFAULT


def _dot(a, b):
    return jnp.dot(a, b, precision=_PREC, preferred_element_type=jnp.float32)


def _body(x_ref, Wr1_ref, br1_ref, Wr2_ref, br2_ref, We1_ref, be1_ref,
          We2_ref, be2_ref, Wm_ref, Wo_ref, bo_ref, out_ref,
          ws_ref, wmo_ref, be2p_ref):
    e = pl.program_id(0)

    bf = lambda a: a.astype(jnp.bfloat16)

    @pl.when(e == 0)
    def _():
        wmo_ref[:] = bf(_dot(bf(Wm_ref[:]), bf(Wo_ref[:])))
        be2p_ref[:] = _dot(bf(be2_ref[:]), wmo_ref[:])
        h = jnp.maximum(_dot(bf(x_ref[:]), bf(Wr1_ref[:])) + br1_ref[:], 0.0)
        logits = _dot(bf(h), bf(Wr2_ref[:])) + br2_ref[:]
        m = jnp.max(logits, axis=1, keepdims=True)
        p = jnp.exp(logits - m)
        ws = p / jnp.sum(p, axis=1, keepdims=True)
        ws_ref[:] = ws
        out_ref[:] = _dot(ws, be2p_ref[:]) + bo_ref[:]

    w2p = bf(_dot(bf(We2_ref[0]), wmo_ref[:]))
    he = jnp.maximum(bf(_dot(bf(x_ref[:]), bf(We1_ref[0])))
                     + bf(be1_ref[0]), jnp.bfloat16(0.0))
    y = _dot(he, w2p)
    onehot = (jax.lax.broadcasted_iota(jnp.int32, (E, CLAP), 0) == e
              ).astype(jnp.bfloat16)
    wcol = _dot(bf(ws_ref[:]), onehot)
    out_ref[:] += wcol * y


@functools.partial(jax.jit)
def kernel(x, W_r1, b_r1, W_r2, b_r2, W_e1, b_e1, W_e2, b_e2,
           W_model, W_out, b_out):
    f32 = jnp.float32
    bf16 = jnp.bfloat16
    hr = W_r1.shape[1]
    pad = HRP - hr
    Wr1p = jnp.pad(W_r1, ((0, 0), (0, pad)))
    br1p = jnp.pad(b_r1, ((0, pad),)).reshape(1, HRP)
    Wr2p = jnp.pad(W_r2, ((0, pad), (0, 0)))
    br2 = b_r2.reshape(1, E)
    be1 = b_e1.reshape(E, 1, L)
    bo = b_out.reshape(1, CLAP)

    out = pl.pallas_call(
        _body,
        grid=(E,),
        in_specs=[
            pl.BlockSpec((T, D), lambda e: (0, 0)),
            pl.BlockSpec((D, HRP), lambda e: (0, 0)),
            pl.BlockSpec((1, HRP), lambda e: (0, 0)),
            pl.BlockSpec((HRP, E), lambda e: (0, 0)),
            pl.BlockSpec((1, E), lambda e: (0, 0)),
            pl.BlockSpec((1, D, L), lambda e: (e, 0, 0)),
            pl.BlockSpec((1, 1, L), lambda e: (e, 0, 0)),
            pl.BlockSpec((1, L, L), lambda e: (e, 0, 0)),
            pl.BlockSpec((E, L), lambda e: (0, 0)),
            pl.BlockSpec((L, L), lambda e: (0, 0)),
            pl.BlockSpec((L, CLAP), lambda e: (0, 0)),
            pl.BlockSpec((1, CLAP), lambda e: (0, 0)),
        ],
        out_specs=pl.BlockSpec((T, CLAP), lambda e: (0, 0)),
        out_shape=jax.ShapeDtypeStruct((T, CLAP), f32),
        scratch_shapes=[
            pltpu.VMEM((T, E), f32),
            pltpu.VMEM((L, CLAP), bf16),
            pltpu.VMEM((E, CLAP), f32),
        ],
        compiler_params=pltpu.CompilerParams(
            dimension_semantics=("arbitrary",)),
    )(x, Wr1p, br1p, Wr2p, br2, W_e1, be1, W_e2, b_e2, W_model, W_out, bo)

    return (out, jnp.zeros((), f32))
